# baseline (speedup 1.0000x reference)
import functools

import jax
import jax.numpy as jnp
from jax import lax
from jax.experimental import pallas as pl
from jax.experimental.pallas import tpu as pltpu
from jax.experimental.pallas import tpu_sc as plsc

N = 10000
F = 128
E = 320000
NC = 2
NS = 16
NW = NC * NS
B = 32
NBUF = 3
T0 = 87
T1 = 540
CAPR = 272
DCH = 2176
NPAD = 10112
RPS = NPAD // NS

_mesh = plsc.VectorSubcoreMesh(core_axis_name="c", subcore_axis_name="s")


@functools.partial(
    pl.kernel,
    out_type=jax.ShapeDtypeStruct((NW, NPAD // F, F), jnp.float32),
    mesh=_mesh,
    compiler_params=pltpu.CompilerParams(needs_layout_passes=False),
    scratch_types=[
        pltpu.VMEM((DCH,), jnp.int32),
        pltpu.VMEM((NPAD // F, F), jnp.float32),
    ],
)
def _deg_kernel(dst_hbm, out_hbm, dst_v, deg_v):
    c = lax.axis_index("c")
    s = lax.axis_index("s")
    w = c * NS + s
    zero16 = jnp.zeros((16,), jnp.float32)
    one16 = jnp.ones((16,), jnp.float32)

    def zbody(i, carry):
        for k in range(F // 16):
            deg_v[i, pl.ds(k * 16, 16)] = zero16
        return carry

    lax.fori_loop(0, NPAD // F, zbody, 0)

    def body(i, carry):
        idx = dst_v[pl.ds(i * 16, 16)] >> 14
        plsc.addupdate_scatter(deg_v, [idx >> 7, idx & 127], one16)
        return carry

    for ch in range(CAPR * 64 // DCH):
        pltpu.sync_copy(dst_hbm.at[w, pl.ds(ch * DCH, DCH)], dst_v)
        lax.fori_loop(0, DCH // 16, body, 0)
    pltpu.sync_copy(deg_v, out_hbm.at[w])


@functools.partial(
    pl.kernel,
    out_type=jax.ShapeDtypeStruct((NC, NPAD, F), jnp.float32),
    mesh=_mesh,
    scratch_types=[
        pltpu.VMEM((CAPR, 64), jnp.int32),
        pltpu.VMEM((NBUF, B), jnp.int32),
        pltpu.VMEM((NBUF, B), jnp.int32),
        pltpu.VMEM((NBUF * B, F), jnp.float32),
        pltpu.VMEM_SHARED((NPAD, F), jnp.float32),
        [pltpu.SemaphoreType.DMA] * NBUF,
    ],
)
def _segsum_kernel(g_hbm, pk_hbm, out_hbm, pk_v, src_v, dst_v, rows_v,
                   acc, sems):
    rows = [rows_v.at[pl.ds(i * B, B)] for i in range(NBUF)]
    c = lax.axis_index("c")
    s = lax.axis_index("s")
    w = c * NS + s
    nb = jnp.where(c == 0, T0, T1)
    pltpu.sync_copy(pk_hbm.at[w], pk_v)

    zero16 = jnp.zeros((16,), jnp.float32)
    ZR = NBUF * B

    def zbody(r, carry):
        for k in range(F // 16):
            rows_v[r, pl.ds(k * 16, 16)] = zero16
        return carry

    lax.fori_loop(0, ZR, zbody, 0)
    for k in range(RPS // ZR):
        pltpu.sync_copy(rows_v, acc.at[pl.ds(s * RPS + k * ZR, ZR)])
    rem = RPS % ZR
    if rem:
        pltpu.sync_copy(rows_v.at[pl.ds(0, rem)],
                        acc.at[pl.ds(s * RPS + RPS - rem, rem)])
    plsc.subcore_barrier()

    def unpack(j, r):
        for k in range(B // 16):
            p = pk_v[j >> 1, pl.ds((j & 1) * B + k * 16, 16)]
            src_v[r, pl.ds(k * 16, 16)] = p & 16383
            dst_v[r, pl.ds(k * 16, 16)] = p >> 14

    for i in range(NBUF):
        unpack(i, i)
        pltpu.async_copy(g_hbm.at[src_v.at[i]], rows[i], sems[i])

    def body(jj, carry):
        j = jj * NBUF
        for i in range(NBUF):
            pltpu.make_async_copy(g_hbm.at[src_v.at[i]], rows[i], sems[i]).wait()
            pltpu.sync_copy(rows[i], acc.at[dst_v.at[i]], add=True)
            unpack(jnp.minimum(j + NBUF + i, nb - 1), i)
            pltpu.async_copy(g_hbm.at[src_v.at[i]], rows[i], sems[i])
        return carry

    lax.fori_loop(0, nb // NBUF, body, 0)
    for i in range(NBUF):
        pltpu.make_async_copy(g_hbm.at[src_v.at[i]], rows[i], sems[i]).wait()
    plsc.subcore_barrier()
    pltpu.sync_copy(acc.at[pl.ds(s * RPS, RPS)], out_hbm.at[c, pl.ds(s * RPS, RPS)])


def _tc_a_body(x_ref, w1_ref, dinv_ref, h1_ref, g1_ref):
    h = jnp.dot(x_ref[...], w1_ref[...], preferred_element_type=jnp.float32)
    h1_ref[...] = h
    g1_ref[...] = h * dinv_ref[...]


def _tc_b_body(s_ref, h1p_ref, dinv_ref, b1_ref, w2_ref, h1_ref, h2p_ref, g2_ref):
    dv = dinv_ref[...]
    s = s_ref[0, :N, :] + s_ref[1, :N, :]
    h1 = jnp.maximum(dv * s + dv * dv * h1p_ref[...] + b1_ref[...], 0.0)
    h1_ref[...] = h1
    h2 = jnp.dot(h1, w2_ref[...], preferred_element_type=jnp.float32)
    h2p_ref[...] = h2
    g2_ref[...] = h2 * dv


def _tc_c_body(s_ref, h2p_ref, dinv_ref, b2_ref, h1_ref, wr_ref, br_ref, out_ref):
    dv = dinv_ref[...]
    s = s_ref[0, :N, :] + s_ref[1, :N, :]
    h2 = jnp.maximum(dv * s + dv * dv * h2p_ref[...] + b2_ref[...], 0.0)
    wr = wr_ref[...]
    logits = (
        jnp.dot(h1_ref[...], wr[:F, :], preferred_element_type=jnp.float32)
        + jnp.dot(h2, wr[F:, :], preferred_element_type=jnp.float32)
        + br_ref[...]
    )
    m = jnp.max(logits, axis=1, keepdims=True)
    e = jnp.exp(logits - m)
    out_ref[...] = e / jnp.sum(e, axis=1, keepdims=True)


_f32 = jnp.float32

_tc_a = pl.pallas_call(
    _tc_a_body,
    out_shape=(
        jax.ShapeDtypeStruct((N, F), _f32),
        jax.ShapeDtypeStruct((N, F), _f32),
    ),
)

_tc_b = pl.pallas_call(
    _tc_b_body,
    out_shape=(
        jax.ShapeDtypeStruct((N, F), _f32),
        jax.ShapeDtypeStruct((N, F), _f32),
        jax.ShapeDtypeStruct((N, F), _f32),
    ),
)

_tc_c = pl.pallas_call(
    _tc_c_body,
    out_shape=jax.ShapeDtypeStruct((N, F), _f32),
)


def kernel(x, edge_index, W1, b1, W2, b2, Wr, br):
    src = edge_index[0].astype(jnp.int32)
    dst = edge_index[1].astype(jnp.int32)
    flat = src | (dst << 14)
    padv = jnp.int32(N << 14)
    L0, L1, CAP = T0 * B, T1 * B, CAPR * 64
    e0 = NS * L0
    core0 = jnp.concatenate(
        [flat[:e0].reshape(NS, L0), jnp.full((NS, CAP - L0), padv)], axis=1)
    rest = jnp.concatenate(
        [flat[e0:], jnp.full((NS * L1 - (E - e0),), padv)]).reshape(NS, L1)
    core1 = jnp.concatenate([rest, jnp.full((NS, CAP - L1), padv)], axis=1)
    pk = jnp.concatenate([core0, core1], axis=0)
    pk_3d = pk.reshape(NW, CAPR, 64)
    pk_2d = pk.reshape(NW, CAP)

    deg_parts = _deg_kernel(pk_2d)
    deg = jnp.sum(deg_parts, axis=0).reshape(NPAD)[:N] + 1.0
    dinv = lax.rsqrt(deg)[:, None]

    H1, G1 = _tc_a(x, W1, dinv)
    S1 = _segsum_kernel(G1, pk_3d)
    h1, H2, G2 = _tc_b(S1, H1, dinv, b1[None, :], W2)
    S2 = _segsum_kernel(G2, pk_3d)
    return _tc_c(S2, H2, dinv, b2[None, :], h1, Wr, br[None, :])


# speedup vs baseline: 1.3729x; 1.3729x over previous
import functools

import jax
import jax.numpy as jnp
from jax import lax
from jax.experimental import pallas as pl
from jax.experimental.pallas import tpu as pltpu
from jax.experimental.pallas import tpu_sc as plsc

N = 10000
F = 128
E = 320000
NC = 2
NS = 16
NW = NC * NS
B = 32
NBUF = 3
T0 = 393
T1 = 234
CAPR = 208
DCH = 1664
NPAD = 10112
RPS = NPAD // NS

_mesh = plsc.VectorSubcoreMesh(core_axis_name="c", subcore_axis_name="s")


@functools.partial(
    pl.kernel,
    out_type=jax.ShapeDtypeStruct((NW, NPAD // F, F), jnp.float32),
    mesh=_mesh,
    compiler_params=pltpu.CompilerParams(needs_layout_passes=False),
    scratch_types=[
        pltpu.VMEM((DCH,), jnp.int32),
        pltpu.VMEM((NPAD // F, F), jnp.float32),
    ],
)
def _deg_kernel(dst_hbm, out_hbm, dst_v, deg_v):
    c = lax.axis_index("c")
    s = lax.axis_index("s")
    w = c * NS + s
    zero16 = jnp.zeros((16,), jnp.float32)
    one16 = jnp.ones((16,), jnp.float32)

    def zbody(i, carry):
        for k in range(F // 16):
            deg_v[i, pl.ds(k * 16, 16)] = zero16
        return carry

    lax.fori_loop(0, NPAD // F, zbody, 0)

    def body(i, carry):
        idx = dst_v[pl.ds(i * 16, 16)] >> 14
        plsc.addupdate_scatter(deg_v, [idx >> 7, idx & 127], one16)
        return carry

    for ch in range(CAPR * 64 // DCH):
        pltpu.sync_copy(dst_hbm.at[w, pl.ds(ch * DCH, DCH)], dst_v)
        lax.fori_loop(0, DCH // 16, body, 0)
    pltpu.sync_copy(deg_v, out_hbm.at[w])


@functools.partial(
    pl.kernel,
    out_type=jax.ShapeDtypeStruct((NC, NPAD, F), jnp.float32),
    mesh=_mesh,
    scratch_types=[
        pltpu.VMEM((CAPR, 64), jnp.int32),
        pltpu.VMEM((NBUF, B), jnp.int32),
        pltpu.VMEM((NBUF, B), jnp.int32),
        pltpu.VMEM((NBUF * B, F), jnp.float32),
        pltpu.VMEM_SHARED((NPAD, F), jnp.float32),
        [pltpu.SemaphoreType.DMA] * NBUF,
    ],
)
def _segsum_kernel(g_hbm, pk_hbm, out_hbm, pk_v, src_v, dst_v, rows_v,
                   acc, sems):
    rows = [rows_v.at[pl.ds(i * B, B)] for i in range(NBUF)]
    c = lax.axis_index("c")
    s = lax.axis_index("s")
    w = c * NS + s
    nb = jnp.where(c == 0, T0, T1)
    pltpu.sync_copy(pk_hbm.at[w], pk_v)

    zero16 = jnp.zeros((16,), jnp.float32)
    ZR = NBUF * B

    def zbody(r, carry):
        for k in range(F // 16):
            rows_v[r, pl.ds(k * 16, 16)] = zero16
        return carry

    lax.fori_loop(0, ZR, zbody, 0)
    for k in range(RPS // ZR):
        pltpu.sync_copy(rows_v, acc.at[pl.ds(s * RPS + k * ZR, ZR)])
    rem = RPS % ZR
    if rem:
        pltpu.sync_copy(rows_v.at[pl.ds(0, rem)],
                        acc.at[pl.ds(s * RPS + RPS - rem, rem)])
    plsc.subcore_barrier()

    def unpack(j, r):
        for k in range(B // 16):
            p = pk_v[j >> 1, pl.ds((j & 1) * B + k * 16, 16)]
            src_v[r, pl.ds(k * 16, 16)] = p & 16383
            dst_v[r, pl.ds(k * 16, 16)] = p >> 14

    for i in range(NBUF):
        unpack(i, i)
        pltpu.async_copy(g_hbm.at[src_v.at[i]], rows[i], sems[i])

    def body(jj, carry):
        j = jj * NBUF
        for i in range(NBUF):
            pltpu.make_async_copy(g_hbm.at[src_v.at[i]], rows[i], sems[i]).wait()
            pltpu.sync_copy(rows[i], acc.at[dst_v.at[i]], add=True)
            unpack(jnp.minimum(j + NBUF + i, nb - 1), i)
            pltpu.async_copy(g_hbm.at[src_v.at[i]], rows[i], sems[i])
        return carry

    lax.fori_loop(0, nb // NBUF, body, 0)
    for i in range(NBUF):
        pltpu.make_async_copy(g_hbm.at[src_v.at[i]], rows[i], sems[i]).wait()
    plsc.subcore_barrier()
    pltpu.sync_copy(acc.at[pl.ds(s * RPS, RPS)], out_hbm.at[c, pl.ds(s * RPS, RPS)])


def _tc_a_body(x_ref, w1_ref, dinv_ref, h1_ref, g1_ref):
    h = jnp.dot(x_ref[...], w1_ref[...], preferred_element_type=jnp.float32)
    h1_ref[...] = h
    g1_ref[...] = h * dinv_ref[...]


def _tc_b_body(s_ref, h1p_ref, dinv_ref, b1_ref, w2_ref, h1_ref, h2p_ref, g2_ref):
    dv = dinv_ref[...]
    s = s_ref[0, :N, :] + s_ref[1, :N, :]
    h1 = jnp.maximum(dv * s + dv * dv * h1p_ref[...] + b1_ref[...], 0.0)
    h1_ref[...] = h1
    h2 = jnp.dot(h1, w2_ref[...], preferred_element_type=jnp.float32)
    h2p_ref[...] = h2
    g2_ref[...] = h2 * dv


def _tc_c_body(s_ref, h2p_ref, dinv_ref, b2_ref, h1_ref, wr_ref, br_ref, out_ref):
    dv = dinv_ref[...]
    s = s_ref[0, :N, :] + s_ref[1, :N, :]
    h2 = jnp.maximum(dv * s + dv * dv * h2p_ref[...] + b2_ref[...], 0.0)
    wr = wr_ref[...]
    logits = (
        jnp.dot(h1_ref[...], wr[:F, :], preferred_element_type=jnp.float32)
        + jnp.dot(h2, wr[F:, :], preferred_element_type=jnp.float32)
        + br_ref[...]
    )
    m = jnp.max(logits, axis=1, keepdims=True)
    e = jnp.exp(logits - m)
    out_ref[...] = e / jnp.sum(e, axis=1, keepdims=True)


_f32 = jnp.float32

_tc_a = pl.pallas_call(
    _tc_a_body,
    out_shape=(
        jax.ShapeDtypeStruct((N, F), _f32),
        jax.ShapeDtypeStruct((N, F), _f32),
    ),
)

_tc_b = pl.pallas_call(
    _tc_b_body,
    out_shape=(
        jax.ShapeDtypeStruct((N, F), _f32),
        jax.ShapeDtypeStruct((N, F), _f32),
        jax.ShapeDtypeStruct((N, F), _f32),
    ),
)

_tc_c = pl.pallas_call(
    _tc_c_body,
    out_shape=jax.ShapeDtypeStruct((N, F), _f32),
)


def kernel(x, edge_index, W1, b1, W2, b2, Wr, br):
    src = edge_index[0].astype(jnp.int32)
    dst = edge_index[1].astype(jnp.int32)
    flat = src | (dst << 14)
    padv = jnp.int32(N << 14)
    L0, L1, CAP = T0 * B, T1 * B, CAPR * 64
    e0 = NS * L0
    core0 = jnp.concatenate(
        [flat[:e0].reshape(NS, L0), jnp.full((NS, CAP - L0), padv)], axis=1)
    rest = jnp.concatenate(
        [flat[e0:], jnp.full((NS * L1 - (E - e0),), padv)]).reshape(NS, L1)
    core1 = jnp.concatenate([rest, jnp.full((NS, CAP - L1), padv)], axis=1)
    pk = jnp.concatenate([core0, core1], axis=0)
    pk_3d = pk.reshape(NW, CAPR, 64)
    pk_2d = pk.reshape(NW, CAP)

    deg_parts = _deg_kernel(pk_2d)
    deg = jnp.sum(deg_parts, axis=0).reshape(NPAD)[:N] + 1.0
    dinv = lax.rsqrt(deg)[:, None]

    H1, G1 = _tc_a(x, W1, dinv)
    S1 = _segsum_kernel(G1, pk_3d)
    h1, H2, G2 = _tc_b(S1, H1, dinv, b1[None, :], W2)
    S2 = _segsum_kernel(G2, pk_3d)
    return _tc_c(S2, H2, dinv, b2[None, :], h1, Wr, br[None, :])
